# baseline (speedup 1.0000x reference)
import functools

import jax
import jax.numpy as jnp
from jax import lax
from jax.experimental import pallas as pl
from jax.experimental.pallas import tpu as pltpu
from jax.experimental.pallas import tpu_sc as plsc

VOCAB = 1000000
EMB = 64
N_TOK = 1024 * 200

_NC = 2
_NS = 16
_NW = _NC * _NS

_PER_W = N_TOK // _NW
_CHUNK = 800
_NCHUNK = _PER_W // _CHUNK

_TBLK = 12288
_WIN = 2 * _TBLK
_NTB = -(-VOCAB // _WIN)
_QROWS = _NTB * _TBLK


def _transpose_body(x_ref, out_ref):
    eye = jnp.eye(EMB, dtype=jnp.bfloat16)

    def tp(k):
        x = x_ref[:, k * _TBLK:(k + 1) * _TBLK]
        b0 = x.astype(jnp.bfloat16)
        r0 = x - b0.astype(jnp.float32)
        b1 = r0.astype(jnp.bfloat16)
        b2 = (r0 - b1.astype(jnp.float32)).astype(jnp.bfloat16)
        dims = (((0,), (0,)), ((), ()))
        y = lax.dot_general(b0, eye, dims, preferred_element_type=jnp.float32)
        y = y + lax.dot_general(b1, eye, dims,
                                preferred_element_type=jnp.float32)
        return y + lax.dot_general(b2, eye, dims,
                                   preferred_element_type=jnp.float32)

    out_ref[...] = jnp.concatenate([tp(0), tp(1)], axis=1)


def _emb_body(src_hbm, table_hbm, out_hbm, idx_v, buf0, buf1, gsem0, gsem1,
              wsem0, wsem1):
    wid = lax.axis_index("s") * _NC + lax.axis_index("c")
    base = wid * _PER_W
    pltpu.sync_copy(src_hbm.at[pl.ds(base, _PER_W)], idx_v)

    bufs = (buf0, buf1)
    gsems = (gsem0, gsem1)
    wsems = (wsem0, wsem1)

    def gather(c):
        b = c % 2
        return pltpu.async_copy(
            table_hbm.at[idx_v.at[pl.ds(c * _CHUNK, _CHUNK)]], bufs[b],
            gsems[b])

    def write(c):
        b = c % 2
        return pltpu.async_copy(
            bufs[b], out_hbm.at[pl.ds(base + c * _CHUNK, _CHUNK)], wsems[b])

    g = [None] * _NCHUNK
    w = [None] * _NCHUNK
    g[0] = gather(0)
    g[1] = gather(1)
    for c in range(_NCHUNK):
        g[c].wait()
        w[c] = write(c)
        if c + 2 < _NCHUNK:
            w[c].wait()
            g[c + 2] = gather(c + 2)
    w[_NCHUNK - 2].wait()
    w[_NCHUNK - 1].wait()


@jax.jit
def _embedding_lookup(src_flat, table):
    table_pairs = pl.pallas_call(
        _transpose_body,
        grid=(_NTB,),
        in_specs=[pl.BlockSpec((EMB, _WIN), lambda i: (0, i))],
        out_specs=pl.BlockSpec((_TBLK, 128), lambda i: (i, 0)),
        out_shape=jax.ShapeDtypeStruct((_QROWS, 128), jnp.float32),
    )(table.T)
    table_rm = table_pairs.reshape(2 * _QROWS, EMB)

    mesh = plsc.VectorSubcoreMesh(core_axis_name="c", subcore_axis_name="s")
    fn = functools.partial(
        pl.kernel,
        mesh=mesh,
        out_type=jax.ShapeDtypeStruct((N_TOK, EMB), jnp.float32),
        scratch_types=[
            pltpu.VMEM((_PER_W,), jnp.int32),
            pltpu.VMEM((_CHUNK, EMB), jnp.float32),
            pltpu.VMEM((_CHUNK, EMB), jnp.float32),
            pltpu.SemaphoreType.DMA,
            pltpu.SemaphoreType.DMA,
            pltpu.SemaphoreType.DMA,
            pltpu.SemaphoreType.DMA,
        ],
        compiler_params=pltpu.CompilerParams(use_tc_tiling_on_sc=False),
    )(_emb_body)
    return fn(src_flat, table_rm)


def kernel(src, seg, table):
    del seg
    v = src.reshape(-1).astype(jnp.int32)
    off = v % _WIN
    part = off // _TBLK
    r = (v // _WIN) * _TBLK + off % _TBLK
    idx = 2 * r + part
    out = _embedding_lookup(idx, table)
    return out.reshape(src.shape[0], src.shape[1], EMB)


# speedup vs baseline: 1.1783x; 1.1783x over previous
import functools

import jax
import jax.numpy as jnp
from jax import lax
from jax.experimental import pallas as pl
from jax.experimental.pallas import tpu as pltpu
from jax.experimental.pallas import tpu_sc as plsc

VOCAB = 1000000
EMB = 64
N_TOK = 1024 * 200

_NC = 2
_NS = 16
_NW = _NC * _NS

_PER_W = N_TOK // _NW
_CHUNK = 800
_NCHUNK = _PER_W // _CHUNK

_TBLK = 16384
_HALF = 524288
_NTB = _HALF // _TBLK
_IN_BLKS = -(-VOCAB // _TBLK)


def _transpose_body(lo_ref, hi_ref, out_ref):
    ylo = jnp.transpose(lo_ref[...], (1, 0))
    yhi = jnp.transpose(hi_ref[...], (1, 0))
    out_ref[...] = jnp.concatenate([ylo, yhi], axis=1)


def _emb_body(src_hbm, table_hbm, out_hbm, idx_v, buf0, buf1, gsem0, gsem1,
              wsem0, wsem1):
    wid = lax.axis_index("s") * _NC + lax.axis_index("c")
    base = wid * _PER_W
    pltpu.sync_copy(src_hbm.at[pl.ds(base, _PER_W)], idx_v)

    bufs = (buf0, buf1)
    gsems = (gsem0, gsem1)
    wsems = (wsem0, wsem1)

    def gather(c):
        b = c % 2
        return pltpu.async_copy(
            table_hbm.at[idx_v.at[pl.ds(c * _CHUNK, _CHUNK)]], bufs[b],
            gsems[b])

    def write(c):
        b = c % 2
        return pltpu.async_copy(
            bufs[b], out_hbm.at[pl.ds(base + c * _CHUNK, _CHUNK)], wsems[b])

    g = [None] * _NCHUNK
    w = [None] * _NCHUNK
    g[0] = gather(0)
    g[1] = gather(1)
    for c in range(_NCHUNK):
        g[c].wait()
        w[c] = write(c)
        if c + 2 < _NCHUNK:
            w[c].wait()
            g[c + 2] = gather(c + 2)
    w[_NCHUNK - 2].wait()
    w[_NCHUNK - 1].wait()


@jax.jit
def _embedding_lookup(src_flat, table):
    table_pairs = pl.pallas_call(
        _transpose_body,
        grid=(_NTB,),
        in_specs=[
            pl.BlockSpec((EMB, _TBLK), lambda i: (0, i)),
            pl.BlockSpec((EMB, _TBLK),
                         lambda i: (0, jnp.minimum(i + _NTB, _IN_BLKS - 1))),
        ],
        out_specs=pl.BlockSpec((_TBLK, 2 * EMB), lambda i: (i, 0)),
        out_shape=jax.ShapeDtypeStruct((_HALF, 2 * EMB), jnp.float32),
    )(table.T, table.T)
    table_rm = table_pairs.reshape(2 * _HALF, EMB)

    mesh = plsc.VectorSubcoreMesh(core_axis_name="c", subcore_axis_name="s")
    fn = functools.partial(
        pl.kernel,
        mesh=mesh,
        out_type=jax.ShapeDtypeStruct((N_TOK, EMB), jnp.float32),
        scratch_types=[
            pltpu.VMEM((_PER_W,), jnp.int32),
            pltpu.VMEM((_CHUNK, EMB), jnp.float32),
            pltpu.VMEM((_CHUNK, EMB), jnp.float32),
            pltpu.SemaphoreType.DMA,
            pltpu.SemaphoreType.DMA,
            pltpu.SemaphoreType.DMA,
            pltpu.SemaphoreType.DMA,
        ],
        compiler_params=pltpu.CompilerParams(use_tc_tiling_on_sc=False),
    )(_emb_body)
    return fn(src_flat, table_rm)


def kernel(src, seg, table):
    del seg
    src_flat = src.reshape(-1).astype(jnp.int32)
    src_flat = jnp.where(src_flat < _HALF, 2 * src_flat,
                         2 * (src_flat - _HALF) + 1)
    out = _embedding_lookup(src_flat, table)
    return out.reshape(src.shape[0], src.shape[1], EMB)
